# baseline (speedup 1.0000x reference)
import functools

import numpy as np
import jax
import jax.numpy as jnp
from jax import lax
from jax.experimental import pallas as pl
from jax.experimental.pallas import tpu as pltpu
from jax.experimental.pallas import tpu_sc as plsc

_R1SQ = np.float32(0.2 * 0.2)
_R2SQ = np.float32(0.4 * 0.4)
_NB = 64
_NC, _NS = 2, 16
_NW = _NC * _NS
_CAP = 960
_SCU = 4
_BUF = _CAP + 16 * _SCU + 160


def _fps_call(px, py, pz, S):
    R, C = px.shape

    def body(px_ref, py_ref, pz_ref, ox_ref, oy_ref, oz_ref):
        x = px_ref[...]
        y = py_ref[...]
        z = pz_ref[...]
        flat = (lax.broadcasted_iota(jnp.int32, (R, C), 0) * C
                + lax.broadcasted_iota(jnp.int32, (R, C), 1))
        cx = x[0, 0]
        cy = y[0, 0]
        cz = z[0, 0]
        ox_ref[0:1, :] = jnp.reshape(cx, (1, 1))
        oy_ref[0:1, :] = jnp.reshape(cy, (1, 1))
        oz_ref[0:1, :] = jnp.reshape(cz, (1, 1))
        d = (x - cx) ** 2 + (y - cy) ** 2 + (z - cz) ** 2

        def step(i, d):
            m = jnp.max(d)
            best = jnp.min(jnp.where(d == m, flat, jnp.int32(R * C)))
            sel = flat == best
            cxk = jnp.sum(jnp.where(sel, x, 0.0))
            cyk = jnp.sum(jnp.where(sel, y, 0.0))
            czk = jnp.sum(jnp.where(sel, z, 0.0))
            ox_ref[pl.ds(i, 1), :] = jnp.reshape(cxk, (1, 1))
            oy_ref[pl.ds(i, 1), :] = jnp.reshape(cyk, (1, 1))
            oz_ref[pl.ds(i, 1), :] = jnp.reshape(czk, (1, 1))
            nd = (x - cxk) ** 2 + (y - cyk) ** 2 + (z - czk) ** 2
            return jnp.minimum(d, nd)

        lax.fori_loop(1, S, step, d)

    f = pl.pallas_call(
        body,
        out_shape=[jax.ShapeDtypeStruct((S, 1), jnp.float32)] * 3)
    return f(px, py, pz)


def _make_sel(S, N, r2, G=0, CSU=_SCU):
    SPT = S // _NW
    NCH = N // 16
    NCLP = 128
    r2bits = jnp.int32(np.float32(r2).view(np.int32))
    ALLBITS = jnp.int32(0x3FFFFFFF)
    mesh = plsc.VectorSubcoreMesh(core_axis_name="c", subcore_axis_name="s",
                                  num_cores=_NC, num_subcores=_NS)
    f32 = jnp.float32
    i32 = jnp.int32

    @functools.partial(
        pl.kernel,
        out_type=[
            jax.ShapeDtypeStruct((S * _NB,), i32),
            jax.ShapeDtypeStruct((S * _NB,), f32),
            jax.ShapeDtypeStruct((S * _NB,), f32),
            jax.ShapeDtypeStruct((S * _NB,), f32),
            jax.ShapeDtypeStruct((S * _NB,), f32),
        ],
        mesh=mesh,
        scratch_types=[
            pltpu.VMEM((N,), f32), pltpu.VMEM((N,), f32), pltpu.VMEM((N,), f32),
            pltpu.VMEM((SPT + 16,), f32), pltpu.VMEM((SPT + 16,), f32),
            pltpu.VMEM((SPT + 16,), f32),
            pltpu.VMEM((_BUF,), i32), pltpu.VMEM((_BUF,), i32),
            pltpu.VMEM((SPT * _NB,), i32),
            pltpu.VMEM((SPT * _NB,), f32), pltpu.VMEM((SPT * _NB,), f32),
            pltpu.VMEM((SPT * _NB,), f32), pltpu.VMEM((SPT * _NB,), f32),
        ] + ([
            pltpu.VMEM((N,), i32),
            pltpu.VMEM((NCLP + 16,), i32),
            pltpu.VMEM((NCLP + 16,), i32),
            pltpu.VMEM((NCLP + 16,), i32),
            pltpu.VMEM((N + 2112,), i32),
            pltpu.VMEM((SPT + 16,), i32),
            pltpu.VMEM((SPT + 16,), i32),
            pltpu.VMEM((SPT + 16,), i32),
        ] if G else []),
        compiler_params=pltpu.CompilerParams(needs_layout_passes=False))
    def sel(candx_h, candy_h, candz_h, centx_h, centy_h, centz_h,
            nbr_h, rx_h, ry_h, rz_h, va_h,
            cxs, cys, czs, ctx, cty, ctz, bbuf, ibuf,
            nbr_st, rx_st, ry_st, rz_st, va_st, *bins):
        if G:
            cellv, lc, ctot, cstart, ids, gxs, gys, gzs = bins
        wid = lax.axis_index("s") * _NC + lax.axis_index("c")
        base = wid * SPT
        pltpu.sync_copy(candx_h, cxs)
        pltpu.sync_copy(candy_h, cys)
        pltpu.sync_copy(candz_h, czs)
        pltpu.sync_copy(centx_h.at[pl.ds(base, SPT)], ctx.at[pl.ds(0, SPT)])
        pltpu.sync_copy(centy_h.at[pl.ds(base, SPT)], cty.at[pl.ds(0, SPT)])
        pltpu.sync_copy(centz_h.at[pl.ds(base, SPT)], ctz.at[pl.ds(0, SPT)])

        iota16 = lax.iota(i32, 16)
        zi16 = jnp.zeros((16,), i32)

        if G:
            fG = f32(G)

            def cid_chunk(j, _):
                xv = cxs[pl.ds(j * 16, 16)]
                yv = cys[pl.ds(j * 16, 16)]
                zv = czs[pl.ds(j * 16, 16)]
                cxi = jnp.clip((xv * fG).astype(i32), 0, G - 1)
                cyi = jnp.clip((yv * fG).astype(i32), 0, G - 1)
                czi = jnp.clip((zv * fG).astype(i32), 0, G - 1)
                cellv[pl.ds(j * 16, 16)] = (cxi * G + cyi) * G + czi
                return 0

            lax.fori_loop(0, N // 16, cid_chunk, 0)

            def gcell_chunk(j, _):
                xv = ctx[pl.ds(j * 16, 16)]
                yv = cty[pl.ds(j * 16, 16)]
                zv = ctz[pl.ds(j * 16, 16)]
                gxs[pl.ds(j * 16, 16)] = jnp.clip((xv * fG).astype(i32), 0, G - 1)
                gys[pl.ds(j * 16, 16)] = jnp.clip((yv * fG).astype(i32), 0, G - 1)
                gzs[pl.ds(j * 16, 16)] = jnp.clip((zv * fG).astype(i32), 0, G - 1)
                return 0

            lax.fori_loop(0, (SPT + 15) // 16, gcell_chunk, 0)

            def zero_lc(j, _):
                lc[pl.ds(j * 16, 16)] = zi16
                return 0

            lax.fori_loop(0, NCLP // 16, zero_lc, 0)

            def cnt_chunk(j, _):
                cv = cellv[pl.ds(j * 16, 16)]
                for q in range(16):
                    c = cv[q]
                    cb = (c // 16) * 16
                    ln = c - cb
                    hit = (iota16 == ln).astype(i32)
                    lc[pl.ds(cb, 16)] = lc[pl.ds(cb, 16)] + hit
                return 0

            lax.fori_loop(0, N // 16, cnt_chunk, 0)

            pc = i32(0)
            for j in range(NCLP // 16):
                chv = lc[j * 16:(j + 1) * 16]
                ctot[j * 16:(j + 1) * 16] = chv
                chp = ((chv + 15) // 16) * 16
                inc = plsc.cumsum(chp)
                cstart[j * 16:(j + 1) * 16] = inc - chp + pc
                pc = pc + inc[15]

            lax.fori_loop(0, NCLP // 16, zero_lc, 0)

            def zero_ids(j, _):
                ids[pl.ds(j * 16, 16)] = zi16
                return 0

            lax.fori_loop(0, (N + 2112) // 16, zero_ids, 0)

            def sc_chunk(j, _):
                cv = cellv[pl.ds(j * 16, 16)]
                for q in range(16):
                    c = cv[q]
                    cb = (c // 16) * 16
                    ln = c - cb
                    hit = iota16 == ln
                    lcv = lc[pl.ds(cb, 16)]
                    slotv = cstart[pl.ds(cb, 16)] + lcv
                    pidv = jnp.full((16,), j * 16 + q, i32)
                    plsc.store_scatter(ids, [slotv], pidv, mask=hit)
                    lc[pl.ds(cb, 16)] = lcv + hit.astype(i32)
                return 0

            lax.fori_loop(0, N // 16, sc_chunk, 0)

        def count_lt(T, ptr):
            def b(kk, cnt):
                for u in range(_SCU):
                    v = bbuf[pl.ds((kk * _SCU + u) * 16, 16)]
                    cnt = cnt + plsc.all_reduce_population_count(v < T)[0]
                return cnt
            return lax.fori_loop(0, (ptr + 16 * _SCU - 1) // (16 * _SCU), b,
                                 i32(0))

        def find64(ptr):
            def bit(t, P):
                C = P | (i32(1) << (29 - t))
                g = count_lt(C, ptr)
                return jnp.where(g <= 63, C, P)
            return lax.fori_loop(0, 30, bit, i32(0))

        PADV = jnp.full((16,), 0x7FFFFFFF, i32)

        def pad_buf(ptr):
            bbuf[pl.ds(ptr, 16)] = PADV
            bbuf[pl.ds(ptr + 16, 16)] = PADV
            bbuf[pl.ds(ptr + 32, 16)] = PADV
            bbuf[pl.ds(ptr + 48, 16)] = PADV

        def compress(ptr):
            X = find64(ptr)

            def b(k, w):
                v = bbuf[pl.ds(k * 16, 16)]
                iv = ibuf[pl.ds(k * 16, 16)]
                keep = v <= X
                plsc.store_compressed(bbuf.at[pl.ds(w, 16)], v, mask=keep)
                plsc.store_compressed(ibuf.at[pl.ds(w, 16)], iv, mask=keep)
                return w + plsc.all_reduce_population_count(keep)[0]

            w = lax.fori_loop(0, (ptr + 15) // 16, b, i32(0))
            pad_buf(w)
            return w, X

        def per_centroid(ci, _):
            ccx = ctx[pl.ds(ci, 16)][0]
            ccy = cty[pl.ds(ci, 16)][0]
            ccz = ctz[pl.ds(ci, 16)][0]

            if not G:
                def ch(jj, carry):
                    ptr, tau = carry
                    for u in range(_SCU):
                        j = jj * _SCU + u
                        xv = cxs[pl.ds(j * 16, 16)]
                        yv = cys[pl.ds(j * 16, 16)]
                        zv = czs[pl.ds(j * 16, 16)]
                        dx = xv - ccx
                        dy = yv - ccy
                        dz = zv - ccz
                        d2 = dx * dx + dy * dy + dz * dz
                        bits = plsc.bitcast(d2, i32)
                        m = bits <= tau
                        plsc.store_compressed(bbuf.at[pl.ds(ptr, 16)], bits,
                                              mask=m)
                        idxv = j * 16 + iota16
                        plsc.store_compressed(ibuf.at[pl.ds(ptr, 16)], idxv,
                                              mask=m)
                        ptr = ptr + plsc.all_reduce_population_count(m)[0]

                    def do_compress():
                        pad_buf(ptr)
                        return compress(ptr)

                    return lax.cond(ptr > _CAP, do_compress, lambda: (ptr, tau))

                ptr, _tau = lax.fori_loop(0, NCH // _SCU, ch, (i32(0), r2bits))
            else:
                gcx = gxs[pl.ds(ci, 16)][0]
                gcy = gys[pl.ds(ci, 16)][0]
                gcz = gzs[pl.ds(ci, 16)][0]

                def cell_scan(cell, carry):
                    st = cstart[pl.ds(cell, 16)][0]
                    cn = ctot[pl.ds(cell, 16)][0]

                    def chb(kk, carry):
                        ptr, tau = carry
                        for u in range(CSU):
                            k = kk * CSU + u
                            lm = (k * 16 + iota16) < cn
                            idv = ids[pl.ds(st + k * 16, 16)]
                            gxv = plsc.load_gather(cxs, [idv])
                            gyv = plsc.load_gather(cys, [idv])
                            gzv = plsc.load_gather(czs, [idv])
                            dx = gxv - ccx
                            dy = gyv - ccy
                            dz = gzv - ccz
                            d2 = dx * dx + dy * dy + dz * dz
                            bits = plsc.bitcast(d2, i32)
                            m = (bits <= tau) & lm
                            plsc.store_compressed(bbuf.at[pl.ds(ptr, 16)],
                                                  bits, mask=m)
                            plsc.store_compressed(ibuf.at[pl.ds(ptr, 16)],
                                                  idv, mask=m)
                            ptr = ptr + plsc.all_reduce_population_count(m)[0]

                        def do_compress():
                            pad_buf(ptr)
                            return compress(ptr)

                        return lax.cond(ptr > _CAP, do_compress,
                                        lambda: (ptr, tau))

                    nsc = (cn + 16 * CSU - 1) // (16 * CSU)
                    return lax.fori_loop(0, nsc, chb, carry)

                def loop_z(oz, carry):
                    return cell_scan((carry[2] * G + carry[3]) * G + oz,
                                     carry[:2]) + carry[2:]

                def loop_y(oy, carry):
                    inner = lax.fori_loop(
                        jnp.maximum(gcz - 1, 0), jnp.minimum(gcz + 1, G - 1) + 1,
                        loop_z, carry[:2] + (carry[2], oy))
                    return inner[:2] + carry[2:]

                def loop_x(ox, carry):
                    inner = lax.fori_loop(
                        jnp.maximum(gcy - 1, 0), jnp.minimum(gcy + 1, G - 1) + 1,
                        loop_y, carry[:2] + (ox,))
                    return inner[:2]

                ptr, _tau = lax.fori_loop(
                    jnp.maximum(gcx - 1, 0), jnp.minimum(gcx + 1, G - 1) + 1,
                    loop_x, (i32(0), r2bits))
            pad_buf(ptr)

            X = lax.cond(ptr <= _NB, lambda: ALLBITS, lambda: find64(ptr))
            needed = i32(_NB) - count_lt(X, ptr)

            def count_eq_lt_idx(T, ptr):
                def b(kk, cnt):
                    for u in range(_SCU):
                        k = kk * _SCU + u
                        v = bbuf[pl.ds(k * 16, 16)]
                        iv = ibuf[pl.ds(k * 16, 16)]
                        msk = (v == X) & (iv < T)
                        cnt = cnt + plsc.all_reduce_population_count(msk)[0]
                    return cnt
                return lax.fori_loop(0, (ptr + 16 * _SCU - 1) // (16 * _SCU),
                                     b, i32(0))

            def find_j():
                def bit(t, P):
                    Cq = P | (i32(1) << (13 - t))
                    g = count_eq_lt_idx(Cq, ptr)
                    return jnp.where(g <= needed - 1, Cq, P)
                return lax.fori_loop(0, 14, bit, i32(0))

            eqcnt = count_eq_lt_idx(i32(0x7FFFFFFF), ptr)
            J = lax.cond(eqcnt > needed, find_j, lambda: i32(0x7FFFFFFF))

            z16f = jnp.zeros((16,), f32)
            z16i = jnp.zeros((16,), i32)
            row = ci * _NB
            for q in range(_NB // 16):
                nbr_st[pl.ds(row + q * 16, 16)] = z16i
                rx_st[pl.ds(row + q * 16, 16)] = z16f
                ry_st[pl.ds(row + q * 16, 16)] = z16f
                rz_st[pl.ds(row + q * 16, 16)] = z16f
                va_st[pl.ds(row + q * 16, 16)] = z16f

            def em(k, optr):
                v = bbuf[pl.ds(k * 16, 16)]
                iv = ibuf[pl.ds(k * 16, 16)]
                sel = (v < X) | ((v == X) & (iv <= J))
                gx = plsc.load_gather(cxs, [iv], mask=sel)
                gy = plsc.load_gather(cys, [iv], mask=sel)
                gz = plsc.load_gather(czs, [iv], mask=sel)
                plsc.store_compressed(nbr_st.at[pl.ds(row + optr, 16)], iv, mask=sel)
                plsc.store_compressed(rx_st.at[pl.ds(row + optr, 16)], gx - ccx, mask=sel)
                plsc.store_compressed(ry_st.at[pl.ds(row + optr, 16)], gy - ccy, mask=sel)
                plsc.store_compressed(rz_st.at[pl.ds(row + optr, 16)], gz - ccz, mask=sel)
                plsc.store_compressed(va_st.at[pl.ds(row + optr, 16)],
                                      jnp.ones((16,), f32), mask=sel)
                return optr + plsc.all_reduce_population_count(sel)[0]

            lax.fori_loop(0, (ptr + 15) // 16, em, i32(0))
            return 0

        lax.fori_loop(0, SPT, per_centroid, 0)
        eb = base * _NB
        en = SPT * _NB
        pltpu.sync_copy(nbr_st, nbr_h.at[pl.ds(eb, en)])
        pltpu.sync_copy(rx_st, rx_h.at[pl.ds(eb, en)])
        pltpu.sync_copy(ry_st, ry_h.at[pl.ds(eb, en)])
        pltpu.sync_copy(rz_st, rz_h.at[pl.ds(eb, en)])
        pltpu.sync_copy(va_st, va_h.at[pl.ds(eb, en)])

    return sel


def _gather_rows(table, idx):
    V, D = table.shape
    B = idx.shape[0]
    bpw = B // _NW
    CH = 256
    nch = bpw // CH
    mesh = plsc.VectorSubcoreMesh(core_axis_name="c", subcore_axis_name="s",
                                  num_cores=_NC, num_subcores=_NS)

    @functools.partial(
        pl.kernel,
        out_type=jax.ShapeDtypeStruct((B, D), jnp.float32),
        mesh=mesh,
        scratch_types=[
            pltpu.VMEM((CH,), jnp.int32),
            pltpu.VMEM((CH, D), jnp.float32),
            pltpu.SemaphoreType.DMA,
        ])
    def k(table_h, idx_h, out_h, idxv, rows, sem):
        wid = lax.axis_index("s") * _NC + lax.axis_index("c")

        def ch(i, _):
            base = wid * bpw + i * CH
            pltpu.sync_copy(idx_h.at[pl.ds(base, CH)], idxv)
            pltpu.async_copy(table_h.at[idxv], rows, sem).wait()
            pltpu.sync_copy(rows, out_h.at[pl.ds(base, CH)])
            return 0

        lax.fori_loop(0, nch, ch, 0)

    return k(table, idx)


def _stats_accum(st_ref, h, w):
    hw = h * w
    s1 = jnp.sum(hw, axis=0)
    s2 = jnp.sum(h * hw, axis=0)
    c = jnp.sum(w)
    F = st_ref.shape[1]

    @pl.when(pl.program_id(0) == 0)
    def _():
        st_ref[...] = jnp.zeros_like(st_ref)

    st_ref[0:1, :] += s1[None, :]
    st_ref[1:2, :] += s2[None, :]
    st_ref[2:3, :] += jnp.broadcast_to(jnp.reshape(c, (1, 1)), (1, F))


def _edge_first(xs, valid, W, b, Be):
    E = xs[0].shape[0]
    F = W.shape[1]
    G = E // Be
    splits = []
    o = 0
    for x in xs:
        splits.append((o, x.shape[1]))
        o += x.shape[1]

    def body(*refs):
        x_refs = refs[:len(xs)]
        val_ref, W_ref, b_ref, h_ref, st_ref = refs[len(xs):]
        h = b_ref[...]
        for (off, d), xr in zip(splits, x_refs):
            h = h + jnp.dot(xr[...], W_ref[off:off + d, :],
                            preferred_element_type=jnp.float32)
        h_ref[...] = h
        _stats_accum(st_ref, h, val_ref[...])

    f = pl.pallas_call(
        body,
        grid=(G,),
        in_specs=[pl.BlockSpec((Be, x.shape[1]), lambda i: (i, 0)) for x in xs]
        + [
            pl.BlockSpec((Be, 1), lambda i: (i, 0)),
            pl.BlockSpec(W.shape, lambda i: (0, 0)),
            pl.BlockSpec((1, F), lambda i: (0, 0)),
        ],
        out_specs=[
            pl.BlockSpec((Be, F), lambda i: (i, 0)),
            pl.BlockSpec((8, F), lambda i: (0, 0)),
        ],
        out_shape=[
            jax.ShapeDtypeStruct((E, F), jnp.float32),
            jax.ShapeDtypeStruct((8, F), jnp.float32),
        ])
    return f(*xs, valid, W, b.reshape(1, F))


def _edge_mid(h_in, valid, scale, shift, W, b, Be):
    E, Din = h_in.shape
    F = W.shape[1]
    G = E // Be

    def body(h_ref, val_ref, sc_ref, sh_ref, W_ref, b_ref, o_ref, st_ref):
        hn = jnp.maximum(h_ref[...] * sc_ref[...] + sh_ref[...], 0.0)
        h = jnp.dot(hn, W_ref[...], preferred_element_type=jnp.float32) + b_ref[...]
        o_ref[...] = h
        _stats_accum(st_ref, h, val_ref[...])

    f = pl.pallas_call(
        body,
        grid=(G,),
        in_specs=[
            pl.BlockSpec((Be, Din), lambda i: (i, 0)),
            pl.BlockSpec((Be, 1), lambda i: (i, 0)),
            pl.BlockSpec((1, Din), lambda i: (0, 0)),
            pl.BlockSpec((1, Din), lambda i: (0, 0)),
            pl.BlockSpec((Din, F), lambda i: (0, 0)),
            pl.BlockSpec((1, F), lambda i: (0, 0)),
        ],
        out_specs=[
            pl.BlockSpec((Be, F), lambda i: (i, 0)),
            pl.BlockSpec((8, F), lambda i: (0, 0)),
        ],
        out_shape=[
            jax.ShapeDtypeStruct((E, F), jnp.float32),
            jax.ShapeDtypeStruct((8, F), jnp.float32),
        ])
    return f(h_in, valid, scale, shift, W, b.reshape(1, F))


def _edge_last(h_in, valid, scale, shift, W, b, Rb):
    E, Din = h_in.shape
    F = W.shape[1]
    R = E // _NB
    Be = Rb * _NB
    G = E // Be

    def body(h_ref, val_ref, sc_ref, sh_ref, W_ref, b_ref, o_ref):
        hn = jnp.maximum(h_ref[...] * sc_ref[...] + sh_ref[...], 0.0)
        h = jnp.dot(hn, W_ref[...], preferred_element_type=jnp.float32) + b_ref[...]
        hm = jnp.where(val_ref[...] > 0.0, h, -jnp.inf)
        h3 = jnp.max(hm.reshape(Rb, _NB, F), axis=1)
        o_ref[...] = jnp.where(h3 > -jnp.inf, h3, 0.0)

    f = pl.pallas_call(
        body,
        grid=(G,),
        in_specs=[
            pl.BlockSpec((Be, Din), lambda i: (i, 0)),
            pl.BlockSpec((Be, 1), lambda i: (i, 0)),
            pl.BlockSpec((1, Din), lambda i: (0, 0)),
            pl.BlockSpec((1, Din), lambda i: (0, 0)),
            pl.BlockSpec((Din, F), lambda i: (0, 0)),
            pl.BlockSpec((1, F), lambda i: (0, 0)),
        ],
        out_specs=pl.BlockSpec((Rb, F), lambda i: (i, 0)),
        out_shape=jax.ShapeDtypeStruct((R, F), jnp.float32))
    return f(h_in, valid, scale, shift, W, b.reshape(1, F))


def _bn_scale_shift(st, g, bb, eps=1e-5):
    cnt = jnp.maximum(st[2, 0], 1.0)
    mean = st[0] / cnt
    var = st[1] / cnt - mean * mean
    rstd = g / jnp.sqrt(var + eps)
    return (rstd.reshape(1, -1), (bb - mean * rstd).reshape(1, -1))


def _final_call(x2, pos2, mlp3, head):
    (W0, b0, g0, bb0), (W1, b1, g1, bb1), (W2, b2) = mlp3
    (H0, hb0), (H1, hb1), (H2, hb2) = head
    W0x, W0p = W0[:x2.shape[1], :], W0[x2.shape[1]:, :]

    def nbn(h, g, bb, eps=1e-5):
        mean = jnp.mean(h, axis=0)
        var = jnp.mean((h - mean) ** 2, axis=0)
        return (h - mean) / jnp.sqrt(var + eps) * g + bb

    def body(x2_ref, p2_ref, W0x_r, W0p_r, b0_r, g0_r, bb0_r,
             W1_r, b1_r, g1_r, bb1_r, W2_r, b2_r,
             H0_r, hb0_r, H1_r, hb1_r, H2_r, hb2_r, o_ref):
        h = (jnp.dot(x2_ref[...], W0x_r[...], preferred_element_type=jnp.float32)
             + jnp.dot(p2_ref[...], W0p_r[...], preferred_element_type=jnp.float32)
             + b0_r[...])
        h = jnp.maximum(nbn(h, g0_r[...], bb0_r[...]), 0.0)
        h = jnp.dot(h, W1_r[...], preferred_element_type=jnp.float32) + b1_r[...]
        h = jnp.maximum(nbn(h, g1_r[...], bb1_r[...]), 0.0)
        h = jnp.dot(h, W2_r[...], preferred_element_type=jnp.float32) + b2_r[...]
        g = jnp.max(h, axis=0, keepdims=True)
        g = jnp.maximum(jnp.dot(g, H0_r[...], preferred_element_type=jnp.float32)
                        + hb0_r[...], 0.0)
        g = jnp.maximum(jnp.dot(g, H1_r[...], preferred_element_type=jnp.float32)
                        + hb1_r[...], 0.0)
        logits = jnp.dot(g, H2_r[...], preferred_element_type=jnp.float32) + hb2_r[...]
        m = jnp.max(logits, axis=-1, keepdims=True)
        sh = logits - m
        o_ref[...] = sh - jnp.log(jnp.sum(jnp.exp(sh), axis=-1, keepdims=True))

    args = [x2, pos2, W0x, W0p, b0.reshape(1, -1), g0.reshape(1, -1),
            bb0.reshape(1, -1), W1, b1.reshape(1, -1), g1.reshape(1, -1),
            bb1.reshape(1, -1), W2, b2.reshape(1, -1),
            H0, hb0.reshape(1, -1), H1, hb1.reshape(1, -1), H2,
            hb2.reshape(1, -1)]
    f = pl.pallas_call(
        body,
        out_shape=jax.ShapeDtypeStruct((1, 10), jnp.float32))
    return f(*args)


def _edge_mlp(xs, valid, layers, Be, Rb):
    (Wa, ba, ga, bba), (Wb, bb_, gb, bbb), (Wc, bc) = layers
    h1, st1 = _edge_first(xs, valid, Wa, ba, Be)
    sc1, sh1 = _bn_scale_shift(st1, ga, bba)
    h2, st2 = _edge_mid(h1, valid, sc1, sh1, Wb, bb_, Be)
    sc2, sh2 = _bn_scale_shift(st2, gb, bbb)
    return _edge_last(h2, valid, sc2, sh2, Wc, bc, Rb)


def _pipeline(pos, params):
    N = pos.shape[0]
    S1, S2 = N // 2, N // 8
    px, py, pz = pos[:, 0], pos[:, 1], pos[:, 2]

    ox1, oy1, oz1 = _fps_call(px.reshape(8, N // 8), py.reshape(8, N // 8),
                              pz.reshape(8, N // 8), S1)
    cx1, cy1, cz1 = ox1[:, 0], oy1[:, 0], oz1[:, 0]
    sel1 = _make_sel(S1, N, _R1SQ, G=5)
    _, rx1, ry1, rz1, v1 = sel1(px, py, pz, cx1, cy1, cz1)
    E1 = S1 * _NB
    valid1 = v1.reshape(E1, 1)
    x1 = _edge_mlp([rx1.reshape(E1, 1), ry1.reshape(E1, 1), rz1.reshape(E1, 1)],
                   valid1, params['mlp1'], Be=8192, Rb=128)

    ox2, oy2, oz2 = _fps_call(cx1.reshape(8, S1 // 8), cy1.reshape(8, S1 // 8),
                              cz1.reshape(8, S1 // 8), S2)
    cx2, cy2, cz2 = ox2[:, 0], oy2[:, 0], oz2[:, 0]
    sel2 = _make_sel(S2, S1, _R2SQ)
    nbr2, rx2, ry2, rz2, v2 = sel2(cx1, cy1, cz1, cx2, cy2, cz2)
    E2 = S2 * _NB
    valid2 = v2.reshape(E2, 1)
    xg = _gather_rows(x1, nbr2.reshape(E2))
    x2 = _edge_mlp([xg, rx2.reshape(E2, 1), ry2.reshape(E2, 1),
                    rz2.reshape(E2, 1)],
                   valid2, params['mlp2'], Be=8192, Rb=128)

    pos2 = jnp.concatenate([ox2, oy2, oz2], axis=1)
    return _final_call(x2, pos2, params['mlp3'], params['head'])


def kernel(pos, batch, params):
    del batch
    return _pipeline(pos, params)


# speedup vs baseline: 1.0269x; 1.0269x over previous
import functools

import numpy as np
import jax
import jax.numpy as jnp
from jax import lax
from jax.experimental import pallas as pl
from jax.experimental.pallas import tpu as pltpu
from jax.experimental.pallas import tpu_sc as plsc

_R1SQ = np.float32(0.2 * 0.2)
_R2SQ = np.float32(0.4 * 0.4)
_NB = 64
_NC, _NS = 2, 16
_NW = _NC * _NS
_CAP = 960
_SCU = 4
_BUF = _CAP + 16 * _SCU + 160


def _fps_call(px, py, pz, S):
    R, C = px.shape

    def body(px_ref, py_ref, pz_ref, ox_ref, oy_ref, oz_ref):
        x = px_ref[...]
        y = py_ref[...]
        z = pz_ref[...]
        flat = (lax.broadcasted_iota(jnp.int32, (R, C), 0) * C
                + lax.broadcasted_iota(jnp.int32, (R, C), 1))
        cx = x[0, 0]
        cy = y[0, 0]
        cz = z[0, 0]
        ox_ref[0:1, :] = jnp.reshape(cx, (1, 1))
        oy_ref[0:1, :] = jnp.reshape(cy, (1, 1))
        oz_ref[0:1, :] = jnp.reshape(cz, (1, 1))
        d = (x - cx) ** 2 + (y - cy) ** 2 + (z - cz) ** 2

        def step(i, d):
            m = jnp.max(d)
            best = jnp.min(jnp.where(d == m, flat, jnp.int32(R * C)))
            sel = flat == best
            cxk = jnp.sum(jnp.where(sel, x, 0.0))
            cyk = jnp.sum(jnp.where(sel, y, 0.0))
            czk = jnp.sum(jnp.where(sel, z, 0.0))
            ox_ref[pl.ds(i, 1), :] = jnp.reshape(cxk, (1, 1))
            oy_ref[pl.ds(i, 1), :] = jnp.reshape(cyk, (1, 1))
            oz_ref[pl.ds(i, 1), :] = jnp.reshape(czk, (1, 1))
            nd = (x - cxk) ** 2 + (y - cyk) ** 2 + (z - czk) ** 2
            return jnp.minimum(d, nd)

        lax.fori_loop(1, S, step, d)

    f = pl.pallas_call(
        body,
        out_shape=[jax.ShapeDtypeStruct((S, 1), jnp.float32)] * 3)
    return f(px, py, pz)


def _make_sel(S, N, r2, G=0, CSU=_SCU):
    SPT = S // _NW
    NCH = N // 16
    NCLP = 128
    r2bits = jnp.int32(np.float32(r2).view(np.int32))
    ALLBITS = jnp.int32(0x3FFFFFFF)
    mesh = plsc.VectorSubcoreMesh(core_axis_name="c", subcore_axis_name="s",
                                  num_cores=_NC, num_subcores=_NS)
    f32 = jnp.float32
    i32 = jnp.int32

    @functools.partial(
        pl.kernel,
        out_type=[
            jax.ShapeDtypeStruct((S * _NB,), i32),
            jax.ShapeDtypeStruct((S * _NB,), f32),
            jax.ShapeDtypeStruct((S * _NB,), f32),
            jax.ShapeDtypeStruct((S * _NB,), f32),
            jax.ShapeDtypeStruct((S * _NB,), f32),
        ],
        mesh=mesh,
        scratch_types=[
            pltpu.VMEM((N,), f32), pltpu.VMEM((N,), f32), pltpu.VMEM((N,), f32),
            pltpu.VMEM((SPT + 16,), f32), pltpu.VMEM((SPT + 16,), f32),
            pltpu.VMEM((SPT + 16,), f32),
            pltpu.VMEM((_BUF,), i32), pltpu.VMEM((_BUF,), i32),
            pltpu.VMEM((SPT * _NB,), i32),
            pltpu.VMEM((SPT * _NB,), f32), pltpu.VMEM((SPT * _NB,), f32),
            pltpu.VMEM((SPT * _NB,), f32), pltpu.VMEM((SPT * _NB,), f32),
        ] + ([
            pltpu.VMEM((N,), i32),
            pltpu.VMEM((NCLP + 16,), i32),
            pltpu.VMEM((NCLP + 16,), i32),
            pltpu.VMEM((NCLP + 16,), i32),
            pltpu.VMEM((N + 2112,), i32),
            pltpu.VMEM((SPT + 16,), i32),
            pltpu.VMEM((SPT + 16,), i32),
            pltpu.VMEM((SPT + 16,), i32),
        ] if G else []),
        compiler_params=pltpu.CompilerParams(needs_layout_passes=False))
    def sel(candx_h, candy_h, candz_h, centx_h, centy_h, centz_h,
            nbr_h, rx_h, ry_h, rz_h, va_h,
            cxs, cys, czs, ctx, cty, ctz, bbuf, ibuf,
            nbr_st, rx_st, ry_st, rz_st, va_st, *bins):
        if G:
            cellv, lc, ctot, cstart, ids, gxs, gys, gzs = bins
        wid = lax.axis_index("s") * _NC + lax.axis_index("c")
        base = wid * SPT
        pltpu.sync_copy(candx_h, cxs)
        pltpu.sync_copy(candy_h, cys)
        pltpu.sync_copy(candz_h, czs)
        pltpu.sync_copy(centx_h.at[pl.ds(base, SPT)], ctx.at[pl.ds(0, SPT)])
        pltpu.sync_copy(centy_h.at[pl.ds(base, SPT)], cty.at[pl.ds(0, SPT)])
        pltpu.sync_copy(centz_h.at[pl.ds(base, SPT)], ctz.at[pl.ds(0, SPT)])

        iota16 = lax.iota(i32, 16)
        zi16 = jnp.zeros((16,), i32)

        if G:
            fG = f32(G)

            def cid_chunk(j, _):
                xv = cxs[pl.ds(j * 16, 16)]
                yv = cys[pl.ds(j * 16, 16)]
                zv = czs[pl.ds(j * 16, 16)]
                cxi = jnp.clip((xv * fG).astype(i32), 0, G - 1)
                cyi = jnp.clip((yv * fG).astype(i32), 0, G - 1)
                czi = jnp.clip((zv * fG).astype(i32), 0, G - 1)
                cellv[pl.ds(j * 16, 16)] = (cxi * G + cyi) * G + czi
                return 0

            lax.fori_loop(0, N // 16, cid_chunk, 0)

            def gcell_chunk(j, _):
                xv = ctx[pl.ds(j * 16, 16)]
                yv = cty[pl.ds(j * 16, 16)]
                zv = ctz[pl.ds(j * 16, 16)]
                gxs[pl.ds(j * 16, 16)] = jnp.clip((xv * fG).astype(i32), 0, G - 1)
                gys[pl.ds(j * 16, 16)] = jnp.clip((yv * fG).astype(i32), 0, G - 1)
                gzs[pl.ds(j * 16, 16)] = jnp.clip((zv * fG).astype(i32), 0, G - 1)
                return 0

            lax.fori_loop(0, (SPT + 15) // 16, gcell_chunk, 0)

            def zero_lc(j, _):
                lc[pl.ds(j * 16, 16)] = zi16
                return 0

            lax.fori_loop(0, NCLP // 16, zero_lc, 0)

            def cnt_chunk(j, _):
                cv = cellv[pl.ds(j * 16, 16)]
                for q in range(16):
                    c = cv[q]
                    cb = (c // 16) * 16
                    ln = c - cb
                    hit = (iota16 == ln).astype(i32)
                    lc[pl.ds(cb, 16)] = lc[pl.ds(cb, 16)] + hit
                return 0

            lax.fori_loop(0, N // 16, cnt_chunk, 0)

            pc = i32(0)
            for j in range(NCLP // 16):
                chv = lc[j * 16:(j + 1) * 16]
                ctot[j * 16:(j + 1) * 16] = chv
                chp = ((chv + 15) // 16) * 16
                inc = plsc.cumsum(chp)
                cstart[j * 16:(j + 1) * 16] = inc - chp + pc
                pc = pc + inc[15]

            lax.fori_loop(0, NCLP // 16, zero_lc, 0)

            def zero_ids(j, _):
                ids[pl.ds(j * 16, 16)] = zi16
                return 0

            lax.fori_loop(0, (N + 2112) // 16, zero_ids, 0)

            def sc_chunk(j, _):
                cv = cellv[pl.ds(j * 16, 16)]
                for q in range(16):
                    c = cv[q]
                    cb = (c // 16) * 16
                    ln = c - cb
                    hit = iota16 == ln
                    lcv = lc[pl.ds(cb, 16)]
                    slotv = cstart[pl.ds(cb, 16)] + lcv
                    pidv = jnp.full((16,), j * 16 + q, i32)
                    plsc.store_scatter(ids, [slotv], pidv, mask=hit)
                    lc[pl.ds(cb, 16)] = lcv + hit.astype(i32)
                return 0

            lax.fori_loop(0, N // 16, sc_chunk, 0)

        def count_lt(T, ptr):
            def b(kk, cnt):
                for u in range(_SCU):
                    v = bbuf[pl.ds((kk * _SCU + u) * 16, 16)]
                    cnt = cnt + plsc.all_reduce_population_count(v < T)
                return cnt
            return lax.fori_loop(0, (ptr + 16 * _SCU - 1) // (16 * _SCU), b,
                                 zi16)[0]

        def find64(ptr):
            def bit(t, P):
                C = P | (i32(1) << (29 - t))
                g = count_lt(C, ptr)
                return jnp.where(g <= 63, C, P)
            return lax.fori_loop(0, 30, bit, i32(0))

        PADV = jnp.full((16,), 0x7FFFFFFF, i32)

        def pad_buf(ptr):
            bbuf[pl.ds(ptr, 16)] = PADV
            bbuf[pl.ds(ptr + 16, 16)] = PADV
            bbuf[pl.ds(ptr + 32, 16)] = PADV
            bbuf[pl.ds(ptr + 48, 16)] = PADV

        def compress(ptr):
            X = find64(ptr)

            def b(k, w):
                v = bbuf[pl.ds(k * 16, 16)]
                iv = ibuf[pl.ds(k * 16, 16)]
                keep = v <= X
                plsc.store_compressed(bbuf.at[pl.ds(w, 16)], v, mask=keep)
                plsc.store_compressed(ibuf.at[pl.ds(w, 16)], iv, mask=keep)
                return w + plsc.all_reduce_population_count(keep)[0]

            w = lax.fori_loop(0, (ptr + 15) // 16, b, i32(0))
            pad_buf(w)
            return w, X

        def per_centroid(ci, _):
            ccx = ctx[pl.ds(ci, 16)][0]
            ccy = cty[pl.ds(ci, 16)][0]
            ccz = ctz[pl.ds(ci, 16)][0]

            if not G:
                def ch(jj, carry):
                    ptr, tau = carry
                    for u in range(_SCU):
                        j = jj * _SCU + u
                        xv = cxs[pl.ds(j * 16, 16)]
                        yv = cys[pl.ds(j * 16, 16)]
                        zv = czs[pl.ds(j * 16, 16)]
                        dx = xv - ccx
                        dy = yv - ccy
                        dz = zv - ccz
                        d2 = dx * dx + dy * dy + dz * dz
                        bits = plsc.bitcast(d2, i32)
                        m = bits <= tau
                        plsc.store_compressed(bbuf.at[pl.ds(ptr, 16)], bits,
                                              mask=m)
                        idxv = j * 16 + iota16
                        plsc.store_compressed(ibuf.at[pl.ds(ptr, 16)], idxv,
                                              mask=m)
                        ptr = ptr + plsc.all_reduce_population_count(m)[0]

                    def do_compress():
                        pad_buf(ptr)
                        return compress(ptr)

                    return lax.cond(ptr > _CAP, do_compress, lambda: (ptr, tau))

                ptr, _tau = lax.fori_loop(0, NCH // _SCU, ch, (i32(0), r2bits))
            else:
                gcx = gxs[pl.ds(ci, 16)][0]
                gcy = gys[pl.ds(ci, 16)][0]
                gcz = gzs[pl.ds(ci, 16)][0]

                def cell_scan(cell, carry):
                    st = cstart[pl.ds(cell, 16)][0]
                    cn = ctot[pl.ds(cell, 16)][0]

                    def chb(kk, carry):
                        ptr, tau = carry
                        for u in range(CSU):
                            k = kk * CSU + u
                            lm = (k * 16 + iota16) < cn
                            idv = ids[pl.ds(st + k * 16, 16)]
                            gxv = plsc.load_gather(cxs, [idv])
                            gyv = plsc.load_gather(cys, [idv])
                            gzv = plsc.load_gather(czs, [idv])
                            dx = gxv - ccx
                            dy = gyv - ccy
                            dz = gzv - ccz
                            d2 = dx * dx + dy * dy + dz * dz
                            bits = plsc.bitcast(d2, i32)
                            m = (bits <= tau) & lm
                            plsc.store_compressed(bbuf.at[pl.ds(ptr, 16)],
                                                  bits, mask=m)
                            plsc.store_compressed(ibuf.at[pl.ds(ptr, 16)],
                                                  idv, mask=m)
                            ptr = ptr + plsc.all_reduce_population_count(m)[0]

                        def do_compress():
                            pad_buf(ptr)
                            return compress(ptr)

                        return lax.cond(ptr > _CAP, do_compress,
                                        lambda: (ptr, tau))

                    nsc = (cn + 16 * CSU - 1) // (16 * CSU)
                    return lax.fori_loop(0, nsc, chb, carry)

                def loop_z(oz, carry):
                    return cell_scan((carry[2] * G + carry[3]) * G + oz,
                                     carry[:2]) + carry[2:]

                def loop_y(oy, carry):
                    inner = lax.fori_loop(
                        jnp.maximum(gcz - 1, 0), jnp.minimum(gcz + 1, G - 1) + 1,
                        loop_z, carry[:2] + (carry[2], oy))
                    return inner[:2] + carry[2:]

                def loop_x(ox, carry):
                    inner = lax.fori_loop(
                        jnp.maximum(gcy - 1, 0), jnp.minimum(gcy + 1, G - 1) + 1,
                        loop_y, carry[:2] + (ox,))
                    return inner[:2]

                ptr, _tau = lax.fori_loop(
                    jnp.maximum(gcx - 1, 0), jnp.minimum(gcx + 1, G - 1) + 1,
                    loop_x, (i32(0), r2bits))
            pad_buf(ptr)

            X = lax.cond(ptr <= _NB, lambda: ALLBITS, lambda: find64(ptr))
            needed = i32(_NB) - count_lt(X, ptr)

            def count_eq_lt_idx(T, ptr):
                def b(kk, cnt):
                    for u in range(_SCU):
                        k = kk * _SCU + u
                        v = bbuf[pl.ds(k * 16, 16)]
                        iv = ibuf[pl.ds(k * 16, 16)]
                        msk = (v == X) & (iv < T)
                        cnt = cnt + plsc.all_reduce_population_count(msk)
                    return cnt
                return lax.fori_loop(0, (ptr + 16 * _SCU - 1) // (16 * _SCU),
                                     b, zi16)[0]

            def find_j():
                def bit(t, P):
                    Cq = P | (i32(1) << (13 - t))
                    g = count_eq_lt_idx(Cq, ptr)
                    return jnp.where(g <= needed - 1, Cq, P)
                return lax.fori_loop(0, 14, bit, i32(0))

            eqcnt = count_eq_lt_idx(i32(0x7FFFFFFF), ptr)
            J = lax.cond(eqcnt > needed, find_j, lambda: i32(0x7FFFFFFF))

            z16f = jnp.zeros((16,), f32)
            z16i = jnp.zeros((16,), i32)
            row = ci * _NB
            for q in range(_NB // 16):
                nbr_st[pl.ds(row + q * 16, 16)] = z16i
                rx_st[pl.ds(row + q * 16, 16)] = z16f
                ry_st[pl.ds(row + q * 16, 16)] = z16f
                rz_st[pl.ds(row + q * 16, 16)] = z16f
                va_st[pl.ds(row + q * 16, 16)] = z16f

            def em(k, optr):
                v = bbuf[pl.ds(k * 16, 16)]
                iv = ibuf[pl.ds(k * 16, 16)]
                sel = (v < X) | ((v == X) & (iv <= J))
                gx = plsc.load_gather(cxs, [iv], mask=sel)
                gy = plsc.load_gather(cys, [iv], mask=sel)
                gz = plsc.load_gather(czs, [iv], mask=sel)
                plsc.store_compressed(nbr_st.at[pl.ds(row + optr, 16)], iv, mask=sel)
                plsc.store_compressed(rx_st.at[pl.ds(row + optr, 16)], gx - ccx, mask=sel)
                plsc.store_compressed(ry_st.at[pl.ds(row + optr, 16)], gy - ccy, mask=sel)
                plsc.store_compressed(rz_st.at[pl.ds(row + optr, 16)], gz - ccz, mask=sel)
                plsc.store_compressed(va_st.at[pl.ds(row + optr, 16)],
                                      jnp.ones((16,), f32), mask=sel)
                return optr + plsc.all_reduce_population_count(sel)[0]

            lax.fori_loop(0, (ptr + 15) // 16, em, i32(0))
            return 0

        lax.fori_loop(0, SPT, per_centroid, 0)
        eb = base * _NB
        en = SPT * _NB
        pltpu.sync_copy(nbr_st, nbr_h.at[pl.ds(eb, en)])
        pltpu.sync_copy(rx_st, rx_h.at[pl.ds(eb, en)])
        pltpu.sync_copy(ry_st, ry_h.at[pl.ds(eb, en)])
        pltpu.sync_copy(rz_st, rz_h.at[pl.ds(eb, en)])
        pltpu.sync_copy(va_st, va_h.at[pl.ds(eb, en)])

    return sel


def _gather_rows(table, idx):
    V, D = table.shape
    B = idx.shape[0]
    bpw = B // _NW
    CH = 256
    nch = bpw // CH
    mesh = plsc.VectorSubcoreMesh(core_axis_name="c", subcore_axis_name="s",
                                  num_cores=_NC, num_subcores=_NS)

    @functools.partial(
        pl.kernel,
        out_type=jax.ShapeDtypeStruct((B, D), jnp.float32),
        mesh=mesh,
        scratch_types=[
            pltpu.VMEM((CH,), jnp.int32),
            pltpu.VMEM((CH, D), jnp.float32),
            pltpu.SemaphoreType.DMA,
        ])
    def k(table_h, idx_h, out_h, idxv, rows, sem):
        wid = lax.axis_index("s") * _NC + lax.axis_index("c")

        def ch(i, _):
            base = wid * bpw + i * CH
            pltpu.sync_copy(idx_h.at[pl.ds(base, CH)], idxv)
            pltpu.async_copy(table_h.at[idxv], rows, sem).wait()
            pltpu.sync_copy(rows, out_h.at[pl.ds(base, CH)])
            return 0

        lax.fori_loop(0, nch, ch, 0)

    return k(table, idx)


def _stats_accum(st_ref, h, w):
    hw = h * w
    s1 = jnp.sum(hw, axis=0)
    s2 = jnp.sum(h * hw, axis=0)
    c = jnp.sum(w)
    F = st_ref.shape[1]

    @pl.when(pl.program_id(0) == 0)
    def _():
        st_ref[...] = jnp.zeros_like(st_ref)

    st_ref[0:1, :] += s1[None, :]
    st_ref[1:2, :] += s2[None, :]
    st_ref[2:3, :] += jnp.broadcast_to(jnp.reshape(c, (1, 1)), (1, F))


def _edge_first(xs, valid, W, b, Be):
    E = xs[0].shape[0]
    F = W.shape[1]
    G = E // Be
    splits = []
    o = 0
    for x in xs:
        splits.append((o, x.shape[1]))
        o += x.shape[1]

    def body(*refs):
        x_refs = refs[:len(xs)]
        val_ref, W_ref, b_ref, h_ref, st_ref = refs[len(xs):]
        h = b_ref[...]
        for (off, d), xr in zip(splits, x_refs):
            h = h + jnp.dot(xr[...], W_ref[off:off + d, :],
                            preferred_element_type=jnp.float32)
        h_ref[...] = h
        _stats_accum(st_ref, h, val_ref[...])

    f = pl.pallas_call(
        body,
        grid=(G,),
        in_specs=[pl.BlockSpec((Be, x.shape[1]), lambda i: (i, 0)) for x in xs]
        + [
            pl.BlockSpec((Be, 1), lambda i: (i, 0)),
            pl.BlockSpec(W.shape, lambda i: (0, 0)),
            pl.BlockSpec((1, F), lambda i: (0, 0)),
        ],
        out_specs=[
            pl.BlockSpec((Be, F), lambda i: (i, 0)),
            pl.BlockSpec((8, F), lambda i: (0, 0)),
        ],
        out_shape=[
            jax.ShapeDtypeStruct((E, F), jnp.float32),
            jax.ShapeDtypeStruct((8, F), jnp.float32),
        ])
    return f(*xs, valid, W, b.reshape(1, F))


def _edge_mid(h_in, valid, scale, shift, W, b, Be):
    E, Din = h_in.shape
    F = W.shape[1]
    G = E // Be

    def body(h_ref, val_ref, sc_ref, sh_ref, W_ref, b_ref, o_ref, st_ref):
        hn = jnp.maximum(h_ref[...] * sc_ref[...] + sh_ref[...], 0.0)
        h = jnp.dot(hn, W_ref[...], preferred_element_type=jnp.float32) + b_ref[...]
        o_ref[...] = h
        _stats_accum(st_ref, h, val_ref[...])

    f = pl.pallas_call(
        body,
        grid=(G,),
        in_specs=[
            pl.BlockSpec((Be, Din), lambda i: (i, 0)),
            pl.BlockSpec((Be, 1), lambda i: (i, 0)),
            pl.BlockSpec((1, Din), lambda i: (0, 0)),
            pl.BlockSpec((1, Din), lambda i: (0, 0)),
            pl.BlockSpec((Din, F), lambda i: (0, 0)),
            pl.BlockSpec((1, F), lambda i: (0, 0)),
        ],
        out_specs=[
            pl.BlockSpec((Be, F), lambda i: (i, 0)),
            pl.BlockSpec((8, F), lambda i: (0, 0)),
        ],
        out_shape=[
            jax.ShapeDtypeStruct((E, F), jnp.float32),
            jax.ShapeDtypeStruct((8, F), jnp.float32),
        ])
    return f(h_in, valid, scale, shift, W, b.reshape(1, F))


def _edge_last(h_in, valid, scale, shift, W, b, Rb):
    E, Din = h_in.shape
    F = W.shape[1]
    R = E // _NB
    Be = Rb * _NB
    G = E // Be

    def body(h_ref, val_ref, sc_ref, sh_ref, W_ref, b_ref, o_ref):
        hn = jnp.maximum(h_ref[...] * sc_ref[...] + sh_ref[...], 0.0)
        h = jnp.dot(hn, W_ref[...], preferred_element_type=jnp.float32) + b_ref[...]
        hm = jnp.where(val_ref[...] > 0.0, h, -jnp.inf)
        h3 = jnp.max(hm.reshape(Rb, _NB, F), axis=1)
        o_ref[...] = jnp.where(h3 > -jnp.inf, h3, 0.0)

    f = pl.pallas_call(
        body,
        grid=(G,),
        in_specs=[
            pl.BlockSpec((Be, Din), lambda i: (i, 0)),
            pl.BlockSpec((Be, 1), lambda i: (i, 0)),
            pl.BlockSpec((1, Din), lambda i: (0, 0)),
            pl.BlockSpec((1, Din), lambda i: (0, 0)),
            pl.BlockSpec((Din, F), lambda i: (0, 0)),
            pl.BlockSpec((1, F), lambda i: (0, 0)),
        ],
        out_specs=pl.BlockSpec((Rb, F), lambda i: (i, 0)),
        out_shape=jax.ShapeDtypeStruct((R, F), jnp.float32))
    return f(h_in, valid, scale, shift, W, b.reshape(1, F))


def _bn_scale_shift(st, g, bb, eps=1e-5):
    cnt = jnp.maximum(st[2, 0], 1.0)
    mean = st[0] / cnt
    var = st[1] / cnt - mean * mean
    rstd = g / jnp.sqrt(var + eps)
    return (rstd.reshape(1, -1), (bb - mean * rstd).reshape(1, -1))


def _final_call(x2, pos2, mlp3, head):
    (W0, b0, g0, bb0), (W1, b1, g1, bb1), (W2, b2) = mlp3
    (H0, hb0), (H1, hb1), (H2, hb2) = head
    W0x, W0p = W0[:x2.shape[1], :], W0[x2.shape[1]:, :]

    def nbn(h, g, bb, eps=1e-5):
        mean = jnp.mean(h, axis=0)
        var = jnp.mean((h - mean) ** 2, axis=0)
        return (h - mean) / jnp.sqrt(var + eps) * g + bb

    def body(x2_ref, p2_ref, W0x_r, W0p_r, b0_r, g0_r, bb0_r,
             W1_r, b1_r, g1_r, bb1_r, W2_r, b2_r,
             H0_r, hb0_r, H1_r, hb1_r, H2_r, hb2_r, o_ref):
        h = (jnp.dot(x2_ref[...], W0x_r[...], preferred_element_type=jnp.float32)
             + jnp.dot(p2_ref[...], W0p_r[...], preferred_element_type=jnp.float32)
             + b0_r[...])
        h = jnp.maximum(nbn(h, g0_r[...], bb0_r[...]), 0.0)
        h = jnp.dot(h, W1_r[...], preferred_element_type=jnp.float32) + b1_r[...]
        h = jnp.maximum(nbn(h, g1_r[...], bb1_r[...]), 0.0)
        h = jnp.dot(h, W2_r[...], preferred_element_type=jnp.float32) + b2_r[...]
        g = jnp.max(h, axis=0, keepdims=True)
        g = jnp.maximum(jnp.dot(g, H0_r[...], preferred_element_type=jnp.float32)
                        + hb0_r[...], 0.0)
        g = jnp.maximum(jnp.dot(g, H1_r[...], preferred_element_type=jnp.float32)
                        + hb1_r[...], 0.0)
        logits = jnp.dot(g, H2_r[...], preferred_element_type=jnp.float32) + hb2_r[...]
        m = jnp.max(logits, axis=-1, keepdims=True)
        sh = logits - m
        o_ref[...] = sh - jnp.log(jnp.sum(jnp.exp(sh), axis=-1, keepdims=True))

    args = [x2, pos2, W0x, W0p, b0.reshape(1, -1), g0.reshape(1, -1),
            bb0.reshape(1, -1), W1, b1.reshape(1, -1), g1.reshape(1, -1),
            bb1.reshape(1, -1), W2, b2.reshape(1, -1),
            H0, hb0.reshape(1, -1), H1, hb1.reshape(1, -1), H2,
            hb2.reshape(1, -1)]
    f = pl.pallas_call(
        body,
        out_shape=jax.ShapeDtypeStruct((1, 10), jnp.float32))
    return f(*args)


def _edge_mlp(xs, valid, layers, Be, Rb):
    (Wa, ba, ga, bba), (Wb, bb_, gb, bbb), (Wc, bc) = layers
    h1, st1 = _edge_first(xs, valid, Wa, ba, Be)
    sc1, sh1 = _bn_scale_shift(st1, ga, bba)
    h2, st2 = _edge_mid(h1, valid, sc1, sh1, Wb, bb_, Be)
    sc2, sh2 = _bn_scale_shift(st2, gb, bbb)
    return _edge_last(h2, valid, sc2, sh2, Wc, bc, Rb)


def _pipeline(pos, params):
    N = pos.shape[0]
    S1, S2 = N // 2, N // 8
    px, py, pz = pos[:, 0], pos[:, 1], pos[:, 2]

    ox1, oy1, oz1 = _fps_call(px.reshape(8, N // 8), py.reshape(8, N // 8),
                              pz.reshape(8, N // 8), S1)
    cx1, cy1, cz1 = ox1[:, 0], oy1[:, 0], oz1[:, 0]
    sel1 = _make_sel(S1, N, _R1SQ, G=5)
    _, rx1, ry1, rz1, v1 = sel1(px, py, pz, cx1, cy1, cz1)
    E1 = S1 * _NB
    valid1 = v1.reshape(E1, 1)
    x1 = _edge_mlp([rx1.reshape(E1, 1), ry1.reshape(E1, 1), rz1.reshape(E1, 1)],
                   valid1, params['mlp1'], Be=8192, Rb=128)

    ox2, oy2, oz2 = _fps_call(cx1.reshape(8, S1 // 8), cy1.reshape(8, S1 // 8),
                              cz1.reshape(8, S1 // 8), S2)
    cx2, cy2, cz2 = ox2[:, 0], oy2[:, 0], oz2[:, 0]
    sel2 = _make_sel(S2, S1, _R2SQ)
    nbr2, rx2, ry2, rz2, v2 = sel2(cx1, cy1, cz1, cx2, cy2, cz2)
    E2 = S2 * _NB
    valid2 = v2.reshape(E2, 1)
    xg = _gather_rows(x1, nbr2.reshape(E2))
    x2 = _edge_mlp([xg, rx2.reshape(E2, 1), ry2.reshape(E2, 1),
                    rz2.reshape(E2, 1)],
                   valid2, params['mlp2'], Be=8192, Rb=128)

    pos2 = jnp.concatenate([ox2, oy2, oz2], axis=1)
    return _final_call(x2, pos2, params['mlp3'], params['head'])


def kernel(pos, batch, params):
    del batch
    return _pipeline(pos, params)
